# Initial kernel scaffold; baseline (speedup 1.0000x reference)
import jax
import jax.numpy as jnp
from jax.experimental import pallas as pl


def kernel(sequence, emb, pos, qkv_w, qkv_b, out_w, out_b, ln1_s, ln1_b, ffn_w1, ffn_b1, ffn_w2, ffn_b2, ln2_s, ln2_b, cw1, cb1, cw2, cb2, cw3, cb3, sw1, sb1, sw2, sb2):
    raise NotImplementedError("write your pallas kernel here")


# speedup vs baseline: 1.0167x; 1.0167x over previous
import functools

import jax
import jax.numpy as jnp
from jax.experimental import pallas as pl
from jax.experimental.pallas import tpu as pltpu

D = 1024
L = 12
H = 16
DFF = 4096
S = 384
EPS = 1e-5
TI = 8

_INTERPRET = False


def _layer_norm(x, s, b):
    mu = jnp.mean(x, -1, keepdims=True)
    var = jnp.mean((x - mu) ** 2, -1, keepdims=True)
    return (x - mu) * jax.lax.rsqrt(var + EPS) * s + b


def _encoder_stack(x, params):
    def step(x, p):
        qw, qb, ow, ob, l1s, l1b, w1, b1, w2, b2, l2s, l2b = p
        Bv, Sv, Dv = x.shape
        hd = Dv // H
        qkv = x @ qw.T + qb
        q, k, v = jnp.split(qkv, 3, axis=-1)
        rs = lambda t: t.reshape(Bv, Sv, H, hd).transpose(0, 2, 1, 3)
        q, k, v = rs(q), rs(k), rs(v)
        att = jax.nn.softmax(
            jnp.einsum('bhqd,bhkd->bhqk', q, k) / jnp.sqrt(jnp.float32(hd)),
            axis=-1)
        o = jnp.einsum('bhqk,bhkd->bhqd', att, v).transpose(0, 2, 1, 3)
        o = o.reshape(Bv, Sv, Dv)
        x = _layer_norm(x + o @ ow.T + ob, l1s, l1b)
        f = jax.nn.gelu(x @ w1.T + b1, approximate=False) @ w2.T + b2
        x = _layer_norm(x + f, l2s, l2b)
        return x, None

    x, _ = jax.lax.scan(step, x, params)
    return x


def _prologue_kernel(enc_ref, cw1t_ref, cb1_ref, sw1t_ref, sb1_ref,
                     sw2t_ref, sb2_ref, ai_ref, aj_ref, sec_ref):
    enc = enc_ref[...]
    proj = jnp.dot(enc, cw1t_ref[...], preferred_element_type=jnp.float32)
    ai_ref[...] = proj[:, :D].astype(jnp.bfloat16)
    aj_ref[...] = (proj[:, D:] + cb1_ref[...]).astype(jnp.bfloat16)
    s1 = jnp.maximum(
        jnp.dot(enc, sw1t_ref[...], preferred_element_type=jnp.float32)
        + sb1_ref[...], 0.0)
    sec_ref[...] = (
        jnp.dot(s1, sw2t_ref[...], preferred_element_type=jnp.float32)
        + sb2_ref[...])


def _prologue(enc, cw1, cb1, sw1, sb1, sw2, sb2):
    return pl.pallas_call(
        _prologue_kernel,
        out_shape=(
            jax.ShapeDtypeStruct((S, D), jnp.bfloat16),
            jax.ShapeDtypeStruct((S, D), jnp.bfloat16),
            jax.ShapeDtypeStruct((S, 8), jnp.float32),
        ),
        interpret=_INTERPRET,
        compiler_params=pltpu.CompilerParams(
            vmem_limit_bytes=60 * 1024 * 1024),
    )(enc, jnp.concatenate([cw1[:, :D].T, cw1[:, D:].T], axis=1),
      cb1[None, :], sw1.T, sb1[None, :], sw2.T, sb2[None, :])


def _contact_kernel(ai_ref, aj_ref, w2t_ref, b2_ref, w3_ref, b3_ref, o_ref):
    ai = ai_ref[...]
    aj = aj_ref[...]
    h = jnp.maximum(ai[:, None, :] + aj[None, :, :], jnp.bfloat16(0))
    h = h.reshape(TI * S, D)
    h2 = jax.lax.dot_general(
        w2t_ref[...], h, (((0,), (1,)), ((), ())),
        preferred_element_type=jnp.float32)
    h2 = jnp.maximum(h2 + b2_ref[...], 0.0)
    o = jax.lax.dot_general(
        w3_ref[...], h2, (((1,), (0,)), ((), ())),
        preferred_element_type=jnp.float32)
    o_ref[...] = jax.nn.sigmoid(o + b3_ref[0])[None]


def _contact(ai, aj, cw2, cb2, cw3, cb3):
    nblk = S // TI
    out = pl.pallas_call(
        _contact_kernel,
        grid=(nblk,),
        in_specs=[
            pl.BlockSpec((TI, D), lambda t: (t, 0)),
            pl.BlockSpec((S, D), lambda t: (0, 0)),
            pl.BlockSpec((D, D // 2), lambda t: (0, 0)),
            pl.BlockSpec((D // 2, 1), lambda t: (0, 0)),
            pl.BlockSpec((1, D // 2), lambda t: (0, 0)),
            pl.BlockSpec(memory_space=pltpu.SMEM),
        ],
        out_specs=pl.BlockSpec((1, 1, TI * S), lambda t: (t, 0, 0)),
        out_shape=jax.ShapeDtypeStruct((nblk, 1, TI * S), jnp.float32),
        interpret=_INTERPRET,
        compiler_params=pltpu.CompilerParams(
            dimension_semantics=("parallel",),
            vmem_limit_bytes=60 * 1024 * 1024),
    )(ai, aj, cw2.T.astype(jnp.bfloat16), cb2[:, None], cw3, cb3)
    return out.reshape(S, S)


def kernel(sequence, emb, pos, qkv_w, qkv_b, out_w, out_b, ln1_s, ln1_b,
           ffn_w1, ffn_b1, ffn_w2, ffn_b2, ln2_s, ln2_b,
           cw1, cb1, cw2, cb2, cw3, cb3, sw1, sb1, sw2, sb2):
    Bv, Sv = sequence.shape
    x = emb[sequence] + pos[:, :Sv, :]
    enc = _encoder_stack(x, (qkv_w, qkv_b, out_w, out_b, ln1_s, ln1_b,
                             ffn_w1, ffn_b1, ffn_w2, ffn_b2, ln2_s, ln2_b))
    enc2d = enc.reshape(Sv, D)
    ai, aj, sec = _prologue(enc2d, cw1, cb1, sw1, sb1, sw2, sb2)
    contact = _contact(ai, aj, cw2, cb2, cw3, cb3)
    return contact[None], sec[None]


# speedup vs baseline: 1.2831x; 1.2620x over previous
import functools

import jax
import jax.numpy as jnp
from jax.experimental import pallas as pl
from jax.experimental.pallas import tpu as pltpu

D = 1024
L = 12
NH = 16
HD = 64
DFF = 4096
S = 384
EPS = 1e-5
TI = 8

_INTERPRET = False
_VMEM = 56 * 1024 * 1024


def _ln2d(y, s, b):
    mu = jnp.mean(y, -1, keepdims=True)
    var = jnp.mean((y - mu) ** 2, -1, keepdims=True)
    return (y - mu) * jax.lax.rsqrt(var + EPS) * s + b


def _combine(base, p0, p1, bias, s, b):
    return _ln2d(base + p0 + p1 + bias, s, b)


def _bdot(a_bf, w_bf):
    return jax.lax.dot_general(
        a_bf, w_bf, (((1,), (1,)), ((), ())),
        preferred_element_type=jnp.float32)


def _attn_body(first, x_ref, *refs):
    if first:
        (qkvw_ref, qkvb_ref, outw_ref, xout_ref, ap_ref, curb_ref) = refs
    else:
        (pin_ref, cbias_ref, cs_ref, cb_ref,
         qkvw_ref, qkvb_ref, outw_ref, xout_ref, ap_ref, curb_ref) = refs
    j = pl.program_id(1)

    @pl.when(j == 0)
    def _():
        if first:
            cur = x_ref[...]
        else:
            cur = _combine(x_ref[...], pin_ref[0], pin_ref[1],
                           cbias_ref[0], cs_ref[0], cb_ref[0])
        xout_ref[...] = cur[None]
        curb_ref[...] = cur.astype(jnp.bfloat16)

    cur_bf = curb_ref[...]
    w = qkvw_ref[...].reshape(3 * 256, D).astype(jnp.bfloat16)
    qkv = _bdot(cur_bf, w)
    qb = qkvb_ref[0]
    q = (qkv[:, 0:256] + qb[0][None, :]) * 0.125
    k = qkv[:, 256:512] + qb[1][None, :]
    v = qkv[:, 512:768] + qb[2][None, :]
    qbf = q.astype(jnp.bfloat16)
    kbf = k.astype(jnp.bfloat16)
    vbf = v.astype(jnp.bfloat16)
    outs = []
    for h in range(4):
        sl = slice(HD * h, HD * h + HD)
        lg = _bdot(qbf[:, sl], kbf[:, sl])
        m = jnp.max(lg, -1, keepdims=True)
        e = jnp.exp(lg - m)
        a = e / jnp.sum(e, -1, keepdims=True)
        outs.append(jax.lax.dot_general(
            a.astype(jnp.bfloat16), vbf[:, sl], (((1,), (0,)), ((), ())),
            preferred_element_type=jnp.float32))
    o = jnp.concatenate(outs, axis=1).astype(jnp.bfloat16)
    contrib = _bdot(o, outw_ref[0].astype(jnp.bfloat16))

    @pl.when(j == 0)
    def _():
        ap_ref[...] = contrib[None]

    @pl.when(j > 0)
    def _():
        ap_ref[...] = ap_ref[...] + contrib[None]


def _attn_call(l, x, partials, cbias, cs, cb, qkv_w4, qkv_b3, out_w):
    first = l == 0
    specs = [pl.BlockSpec((S, D), lambda c, j: (0, 0))]
    args = [x]
    if not first:
        specs += [
            pl.BlockSpec((2, S, D), lambda c, j: (0, 0, 0)),
            pl.BlockSpec((1, 1, D), lambda c, j: (l - 1, 0, 0)),
            pl.BlockSpec((1, 1, D), lambda c, j: (l - 1, 0, 0)),
            pl.BlockSpec((1, 1, D), lambda c, j: (l - 1, 0, 0)),
        ]
        args += [partials, cbias, cs, cb]
    specs += [
        pl.BlockSpec((1, 3, 256, D), lambda c, j: (l, 0, 2 * c + j, 0)),
        pl.BlockSpec((1, 3, 256), lambda c, j: (l, 0, 2 * c + j)),
        pl.BlockSpec((1, D, 256), lambda c, j: (l, 0, 2 * c + j)),
    ]
    args += [qkv_w4, qkv_b3, out_w]
    return pl.pallas_call(
        functools.partial(_attn_body, first),
        grid=(2, 2),
        in_specs=specs,
        out_specs=(
            pl.BlockSpec((1, S, D), lambda c, j: (c, 0, 0)),
            pl.BlockSpec((1, S, D), lambda c, j: (c, 0, 0)),
        ),
        out_shape=(
            jax.ShapeDtypeStruct((2, S, D), jnp.float32),
            jax.ShapeDtypeStruct((2, S, D), jnp.float32),
        ),
        scratch_shapes=[pltpu.VMEM((S, D), jnp.bfloat16)],
        interpret=_INTERPRET,
        compiler_params=pltpu.CompilerParams(
            dimension_semantics=("parallel", "arbitrary"),
            vmem_limit_bytes=_VMEM),
    )(*args)


def _ffn_body(x_ref, ap_ref_in, obias_ref, l1s_ref, l1b_ref,
              w1_ref, b1_ref, w2_ref, x1out_ref, fp_ref, x1b_ref):
    j = pl.program_id(1)

    @pl.when(j == 0)
    def _():
        x1 = _combine(x_ref[...], ap_ref_in[0], ap_ref_in[1],
                      obias_ref[0], l1s_ref[0], l1b_ref[0])
        x1out_ref[...] = x1[None]
        x1b_ref[...] = x1.astype(jnp.bfloat16)

    x1_bf = x1b_ref[...]
    h = _bdot(x1_bf, w1_ref[0].astype(jnp.bfloat16))
    h = h + b1_ref[0]
    h = h * 0.5 * (1.0 + jax.lax.erf(h * 0.7071067811865476))
    contrib = _bdot(h.astype(jnp.bfloat16),
                    w2_ref[0].astype(jnp.bfloat16))

    @pl.when(j == 0)
    def _():
        fp_ref[...] = contrib[None]

    @pl.when(j > 0)
    def _():
        fp_ref[...] = fp_ref[...] + contrib[None]


def _ffn_call(l, x, partials, obias, l1s, l1b, ffn_w1, ffn_b1, ffn_w2):
    return pl.pallas_call(
        _ffn_body,
        grid=(2, 4),
        in_specs=[
            pl.BlockSpec((S, D), lambda c, j: (0, 0)),
            pl.BlockSpec((2, S, D), lambda c, j: (0, 0, 0)),
            pl.BlockSpec((1, 1, D), lambda c, j: (l, 0, 0)),
            pl.BlockSpec((1, 1, D), lambda c, j: (l, 0, 0)),
            pl.BlockSpec((1, 1, D), lambda c, j: (l, 0, 0)),
            pl.BlockSpec((1, 512, D), lambda c, j: (l, 4 * c + j, 0)),
            pl.BlockSpec((1, 1, 512), lambda c, j: (l, 0, 4 * c + j)),
            pl.BlockSpec((1, D, 512), lambda c, j: (l, 0, 4 * c + j)),
        ],
        out_specs=(
            pl.BlockSpec((1, S, D), lambda c, j: (c, 0, 0)),
            pl.BlockSpec((1, S, D), lambda c, j: (c, 0, 0)),
        ),
        out_shape=(
            jax.ShapeDtypeStruct((2, S, D), jnp.float32),
            jax.ShapeDtypeStruct((2, S, D), jnp.float32),
        ),
        scratch_shapes=[pltpu.VMEM((S, D), jnp.bfloat16)],
        interpret=_INTERPRET,
        compiler_params=pltpu.CompilerParams(
            dimension_semantics=("parallel", "arbitrary"),
            vmem_limit_bytes=_VMEM),
    )(x, partials, obias, l1s, l1b, ffn_w1, ffn_b1, ffn_w2)


def _prologue_kernel(x1_ref, fp_ref, b2_ref, l2s_ref, l2b_ref,
                     cw1t_ref, cb1_ref, sw1t_ref, sb1_ref,
                     sw2t_ref, sb2_ref, ai_ref, aj_ref, sec_ref):
    enc = _combine(x1_ref[...], fp_ref[0], fp_ref[1],
                   b2_ref[...], l2s_ref[...], l2b_ref[...])
    proj = jnp.dot(enc, cw1t_ref[...], preferred_element_type=jnp.float32)
    ai_ref[...] = proj[:, :D].astype(jnp.bfloat16)
    aj_ref[...] = (proj[:, D:] + cb1_ref[...]).astype(jnp.bfloat16)
    s1 = jnp.maximum(
        jnp.dot(enc, sw1t_ref[...], preferred_element_type=jnp.float32)
        + sb1_ref[...], 0.0)
    sec_ref[...] = (
        jnp.dot(s1, sw2t_ref[...], preferred_element_type=jnp.float32)
        + sb2_ref[...])


def _prologue(x1, fp, b2, l2s, l2b, cw1, cb1, sw1, sb1, sw2, sb2):
    return pl.pallas_call(
        _prologue_kernel,
        out_shape=(
            jax.ShapeDtypeStruct((S, D), jnp.bfloat16),
            jax.ShapeDtypeStruct((S, D), jnp.bfloat16),
            jax.ShapeDtypeStruct((S, 8), jnp.float32),
        ),
        interpret=_INTERPRET,
        compiler_params=pltpu.CompilerParams(vmem_limit_bytes=_VMEM),
    )(x1, fp, b2[None, :], l2s[None, :], l2b[None, :],
      jnp.concatenate([cw1[:, :D].T, cw1[:, D:].T], axis=1),
      cb1[None, :], sw1.T, sb1[None, :], sw2.T, sb2[None, :])


def _contact_kernel(ai_ref, aj_ref, w2t_ref, b2_ref, w3_ref, b3_ref, o_ref):
    ai = ai_ref[...]
    aj = aj_ref[...]
    h = jnp.maximum(ai[:, None, :] + aj[None, :, :], jnp.bfloat16(0))
    h = h.reshape(TI * S, D)
    h2 = jax.lax.dot_general(
        w2t_ref[...], h, (((0,), (1,)), ((), ())),
        preferred_element_type=jnp.float32)
    h2 = jnp.maximum(h2 + b2_ref[...], 0.0)
    o = jax.lax.dot_general(
        w3_ref[...], h2, (((1,), (0,)), ((), ())),
        preferred_element_type=jnp.float32)
    o_ref[...] = jax.nn.sigmoid(o + b3_ref[0])[None]


def _contact(ai, aj, cw2, cb2, cw3, cb3):
    nblk = S // TI
    out = pl.pallas_call(
        _contact_kernel,
        grid=(nblk,),
        in_specs=[
            pl.BlockSpec((TI, D), lambda t: (t, 0)),
            pl.BlockSpec((S, D), lambda t: (0, 0)),
            pl.BlockSpec((D, D // 2), lambda t: (0, 0)),
            pl.BlockSpec((D // 2, 1), lambda t: (0, 0)),
            pl.BlockSpec((1, D // 2), lambda t: (0, 0)),
            pl.BlockSpec(memory_space=pltpu.SMEM),
        ],
        out_specs=pl.BlockSpec((1, 1, TI * S), lambda t: (t, 0, 0)),
        out_shape=jax.ShapeDtypeStruct((nblk, 1, TI * S), jnp.float32),
        interpret=_INTERPRET,
        compiler_params=pltpu.CompilerParams(
            dimension_semantics=("parallel",),
            vmem_limit_bytes=_VMEM),
    )(ai, aj, cw2.T.astype(jnp.bfloat16), cb2[:, None], cw3, cb3)
    return out.reshape(S, S)


def kernel(sequence, emb, pos, qkv_w, qkv_b, out_w, out_b, ln1_s, ln1_b,
           ffn_w1, ffn_b1, ffn_w2, ffn_b2, ln2_s, ln2_b,
           cw1, cb1, cw2, cb2, cw3, cb3, sw1, sb1, sw2, sb2):
    Bv, Sv = sequence.shape
    x = (emb[sequence] + pos[:, :Sv, :]).reshape(Sv, D)
    qkv_w4 = qkv_w.reshape(L, 3, D, D)
    qkv_b3 = qkv_b.reshape(L, 3, D)
    ffn_b1r = ffn_b1.reshape(L, 1, DFF)
    out_b_r = out_b.reshape(L, 1, D)
    ln1_s_r = ln1_s.reshape(L, 1, D)
    ln1_b_r = ln1_b.reshape(L, 1, D)
    ffn_b2_r = ffn_b2.reshape(L, 1, D)
    ln2_s_r = ln2_s.reshape(L, 1, D)
    ln2_b_r = ln2_b.reshape(L, 1, D)

    partials = None
    for l in range(L):
        x, ap = _attn_call(l, x, partials, ffn_b2_r, ln2_s_r, ln2_b_r,
                           qkv_w4, qkv_b3, out_w)
        x = x[0]
        x, fp = _ffn_call(l, x, ap, out_b_r, ln1_s_r, ln1_b_r,
                          ffn_w1, ffn_b1r, ffn_w2)
        x = x[0]
        partials = fp

    ai, aj, sec = _prologue(x, partials, ffn_b2[L - 1], ln2_s[L - 1],
                            ln2_b[L - 1], cw1, cb1, sw1, sb1, sw2, sb2)
    contact = _contact(ai, aj, cw2, cb2, cw3, cb3)
    return contact[None], sec[None]


# speedup vs baseline: 1.5536x; 1.2108x over previous
import functools

import jax
import jax.numpy as jnp
from jax.experimental import pallas as pl
from jax.experimental.pallas import tpu as pltpu

D = 1024
L = 12
NH = 16
HD = 64
DFF = 4096
S = 384
EPS = 1e-5
TI = 8

_INTERPRET = False
_VMEM = 56 * 1024 * 1024


def _ln2d(y, s, b):
    mu = jnp.mean(y, -1, keepdims=True)
    var = jnp.mean((y - mu) ** 2, -1, keepdims=True)
    return (y - mu) * jax.lax.rsqrt(var + EPS) * s + b


def _bdot(a_bf, w_bf):
    return jax.lax.dot_general(
        a_bf, w_bf, (((1,), (1,)), ((), ())),
        preferred_element_type=jnp.float32)


def _layer_body(first, x_ref, *refs):
    if first:
        (qkvw_ref, qkvb_ref, outw_ref, ob_ref, l1s_ref, l1b_ref,
         w1_ref, b1_ref, w2_ref, x1out_ref, fp_ref,
         curf, curbf, x1bf, apacc) = refs
    else:
        (fpin_ref, b2p_ref, l2s_ref, l2b_ref,
         qkvw_ref, qkvb_ref, outw_ref, ob_ref, l1s_ref, l1b_ref,
         w1_ref, b1_ref, w2_ref, x1out_ref, fp_ref,
         curf, curbf, x1bf, apacc) = refs
    t = pl.program_id(0)

    @pl.when(t == 0)
    def _():
        if first:
            cur = x_ref[...]
        else:
            cur = _ln2d(x_ref[...] + fpin_ref[...] + b2p_ref[0],
                        l2s_ref[0], l2b_ref[0])
        curf[...] = cur
        curbf[...] = cur.astype(jnp.bfloat16)

    @pl.when(t < 4)
    def _():
        cur_bf = curbf[...]
        w = qkvw_ref[...].reshape(3 * 256, D).astype(jnp.bfloat16)
        qkv = _bdot(cur_bf, w)
        qb = qkvb_ref[0]
        q = (qkv[:, 0:256] + qb[0][None, :]) * 0.125
        k = qkv[:, 256:512] + qb[1][None, :]
        v = qkv[:, 512:768] + qb[2][None, :]
        qbf = q.astype(jnp.bfloat16)
        kbf = k.astype(jnp.bfloat16)
        vbf = v.astype(jnp.bfloat16)
        outs = []
        for h in range(4):
            sl = slice(HD * h, HD * h + HD)
            lg = _bdot(qbf[:, sl], kbf[:, sl])
            m = jnp.max(lg, -1, keepdims=True)
            e = jnp.exp(lg - m)
            a = e / jnp.sum(e, -1, keepdims=True)
            outs.append(jax.lax.dot_general(
                a.astype(jnp.bfloat16), vbf[:, sl], (((1,), (0,)), ((), ())),
                preferred_element_type=jnp.float32))
        o = jnp.concatenate(outs, axis=1).astype(jnp.bfloat16)
        contrib = _bdot(o, outw_ref[0].astype(jnp.bfloat16))

        @pl.when(t == 0)
        def _():
            apacc[...] = contrib

        @pl.when(t > 0)
        def _():
            apacc[...] = apacc[...] + contrib

    @pl.when(t == 4)
    def _():
        x1 = _ln2d(curf[...] + apacc[...] + ob_ref[0],
                   l1s_ref[0], l1b_ref[0])
        x1out_ref[...] = x1
        x1bf[...] = x1.astype(jnp.bfloat16)

    @pl.when(t >= 4)
    def _():
        h = _bdot(x1bf[...], w1_ref[0].astype(jnp.bfloat16))
        h = h + b1_ref[0]
        h = h * 0.5 * (1.0 + jax.lax.erf(h * 0.7071067811865476))
        contrib = _bdot(h.astype(jnp.bfloat16),
                        w2_ref[0].astype(jnp.bfloat16))

        @pl.when(t == 4)
        def _():
            fp_ref[...] = contrib

        @pl.when(t > 4)
        def _():
            fp_ref[...] = fp_ref[...] + contrib


def _layer_call(l, x, fp_prev, b2p, l2s, l2b,
                qkv_w4, qkv_b3, out_w, out_b, l1s, l1b, w1, b1r, w2):
    first = l == 0
    specs = [pl.BlockSpec((S, D), lambda t: (0, 0))]
    args = [x]
    if not first:
        specs += [
            pl.BlockSpec((S, D), lambda t: (0, 0)),
            pl.BlockSpec((1, 1, D), lambda t: (l - 1, 0, 0)),
            pl.BlockSpec((1, 1, D), lambda t: (l - 1, 0, 0)),
            pl.BlockSpec((1, 1, D), lambda t: (l - 1, 0, 0)),
        ]
        args += [fp_prev, b2p, l2s, l2b]
    specs += [
        pl.BlockSpec((1, 3, 256, D), lambda t: (l, 0, jnp.minimum(t, 3), 0)),
        pl.BlockSpec((1, 3, 256), lambda t: (l, 0, jnp.minimum(t, 3))),
        pl.BlockSpec((1, D, 256), lambda t: (l, 0, jnp.minimum(t, 3))),
        pl.BlockSpec((1, 1, D), lambda t: (l, 0, 0)),
        pl.BlockSpec((1, 1, D), lambda t: (l, 0, 0)),
        pl.BlockSpec((1, 1, D), lambda t: (l, 0, 0)),
        pl.BlockSpec((1, 512, D), lambda t: (l, jnp.maximum(t - 4, 0), 0)),
        pl.BlockSpec((1, 1, 512), lambda t: (l, 0, jnp.maximum(t - 4, 0))),
        pl.BlockSpec((1, D, 512), lambda t: (l, 0, jnp.maximum(t - 4, 0))),
    ]
    args += [qkv_w4, qkv_b3, out_w, out_b, l1s, l1b, w1, b1r, w2]
    return pl.pallas_call(
        functools.partial(_layer_body, first),
        grid=(12,),
        in_specs=specs,
        out_specs=(
            pl.BlockSpec((S, D), lambda t: (0, 0)),
            pl.BlockSpec((S, D), lambda t: (0, 0)),
        ),
        out_shape=(
            jax.ShapeDtypeStruct((S, D), jnp.float32),
            jax.ShapeDtypeStruct((S, D), jnp.float32),
        ),
        scratch_shapes=[
            pltpu.VMEM((S, D), jnp.float32),
            pltpu.VMEM((S, D), jnp.bfloat16),
            pltpu.VMEM((S, D), jnp.bfloat16),
            pltpu.VMEM((S, D), jnp.float32),
        ],
        interpret=_INTERPRET,
        compiler_params=pltpu.CompilerParams(
            dimension_semantics=("arbitrary",),
            vmem_limit_bytes=_VMEM),
    )(*args)


def _prologue_kernel(x1_ref, fp_ref, b2_ref, l2s_ref, l2b_ref,
                     cw1t_ref, cb1_ref, sw1t_ref, sb1_ref,
                     sw2t_ref, sb2_ref, ai_ref, aj_ref, sec_ref):
    enc = _ln2d(x1_ref[...] + fp_ref[...] + b2_ref[...],
                l2s_ref[...], l2b_ref[...])
    proj = jnp.dot(enc, cw1t_ref[...], preferred_element_type=jnp.float32)
    ai_ref[...] = proj[:, :D].astype(jnp.bfloat16)
    aj_ref[...] = (proj[:, D:] + cb1_ref[...]).astype(jnp.bfloat16)
    s1 = jnp.maximum(
        jnp.dot(enc, sw1t_ref[...], preferred_element_type=jnp.float32)
        + sb1_ref[...], 0.0)
    sec_ref[...] = (
        jnp.dot(s1, sw2t_ref[...], preferred_element_type=jnp.float32)
        + sb2_ref[...])


def _prologue(x1, fp, b2, l2s, l2b, cw1, cb1, sw1, sb1, sw2, sb2):
    return pl.pallas_call(
        _prologue_kernel,
        out_shape=(
            jax.ShapeDtypeStruct((S, D), jnp.bfloat16),
            jax.ShapeDtypeStruct((S, D), jnp.bfloat16),
            jax.ShapeDtypeStruct((S, 8), jnp.float32),
        ),
        interpret=_INTERPRET,
        compiler_params=pltpu.CompilerParams(vmem_limit_bytes=_VMEM),
    )(x1, fp, b2[None, :], l2s[None, :], l2b[None, :],
      jnp.concatenate([cw1[:, :D].T, cw1[:, D:].T], axis=1),
      cb1[None, :], sw1.T, sb1[None, :], sw2.T, sb2[None, :])


def _contact_kernel(ai_ref, aj_ref, w2t_ref, b2_ref, w3_ref, b3_ref, o_ref):
    ai = ai_ref[...]
    aj = aj_ref[...]
    h = jnp.maximum(ai[:, None, :] + aj[None, :, :], jnp.bfloat16(0))
    h = h.reshape(TI * S, D)
    h2 = jax.lax.dot_general(
        w2t_ref[...], h, (((0,), (1,)), ((), ())),
        preferred_element_type=jnp.float32)
    h2 = jnp.maximum(h2 + b2_ref[...], 0.0)
    o = jax.lax.dot_general(
        w3_ref[...], h2, (((1,), (0,)), ((), ())),
        preferred_element_type=jnp.float32)
    o_ref[...] = jax.nn.sigmoid(o + b3_ref[0])[None]


def _contact(ai, aj, cw2, cb2, cw3, cb3):
    nblk = S // TI
    out = pl.pallas_call(
        _contact_kernel,
        grid=(nblk,),
        in_specs=[
            pl.BlockSpec((TI, D), lambda t: (t, 0)),
            pl.BlockSpec((S, D), lambda t: (0, 0)),
            pl.BlockSpec((D, D // 2), lambda t: (0, 0)),
            pl.BlockSpec((D // 2, 1), lambda t: (0, 0)),
            pl.BlockSpec((1, D // 2), lambda t: (0, 0)),
            pl.BlockSpec(memory_space=pltpu.SMEM),
        ],
        out_specs=pl.BlockSpec((1, 1, TI * S), lambda t: (t, 0, 0)),
        out_shape=jax.ShapeDtypeStruct((nblk, 1, TI * S), jnp.float32),
        interpret=_INTERPRET,
        compiler_params=pltpu.CompilerParams(
            dimension_semantics=("arbitrary",),
            vmem_limit_bytes=_VMEM),
    )(ai, aj, cw2.T.astype(jnp.bfloat16), cb2[:, None], cw3, cb3)
    return out.reshape(S, S)


def kernel(sequence, emb, pos, qkv_w, qkv_b, out_w, out_b, ln1_s, ln1_b,
           ffn_w1, ffn_b1, ffn_w2, ffn_b2, ln2_s, ln2_b,
           cw1, cb1, cw2, cb2, cw3, cb3, sw1, sb1, sw2, sb2):
    Bv, Sv = sequence.shape
    x = (emb[sequence] + pos[:, :Sv, :]).reshape(Sv, D)
    qkv_w4 = qkv_w.reshape(L, 3, D, D)
    qkv_b3 = qkv_b.reshape(L, 3, D)
    ffn_b1r = ffn_b1.reshape(L, 1, DFF)
    out_b_r = out_b.reshape(L, 1, D)
    ln1_s_r = ln1_s.reshape(L, 1, D)
    ln1_b_r = ln1_b.reshape(L, 1, D)
    ffn_b2_r = ffn_b2.reshape(L, 1, D)
    ln2_s_r = ln2_s.reshape(L, 1, D)
    ln2_b_r = ln2_b.reshape(L, 1, D)

    fp = None
    for l in range(L):
        x, fp = _layer_call(l, x, fp, ffn_b2_r, ln2_s_r, ln2_b_r,
                            qkv_w4, qkv_b3, out_w, out_b_r, ln1_s_r,
                            ln1_b_r, ffn_w1, ffn_b1r, ffn_w2)

    ai, aj, sec = _prologue(x, fp, ffn_b2[L - 1], ln2_s[L - 1],
                            ln2_b[L - 1], cw1, cb1, sw1, sb1, sw2, sb2)
    contact = _contact(ai, aj, cw2, cb2, cw3, cb3)
    return contact[None], sec[None]
